# Initial kernel scaffold; baseline (speedup 1.0000x reference)
import jax
import jax.numpy as jnp
from jax.experimental import pallas as pl


def kernel(x, weight):
    raise NotImplementedError("write your pallas kernel here")


# speedup vs baseline: 1.0947x; 1.0947x over previous
import functools

import jax
import jax.numpy as jnp
from jax import lax
from jax.experimental import pallas as pl
from jax.experimental.pallas import tpu as pltpu
from jax.experimental.pallas import tpu_sc as plsc

_B = 16384 * 50
_D = 32
_NW = 32
_BPW = _B // _NW
_C = 1024
_NCHUNK = _BPW // _C

_mesh = plsc.VectorSubcoreMesh(core_axis_name="c", subcore_axis_name="s")


@functools.partial(
    pl.kernel,
    mesh=_mesh,
    out_type=jax.ShapeDtypeStruct((_B, _D), jnp.float32),
    compiler_params=pltpu.CompilerParams(use_tc_tiling_on_sc=False),
    scratch_types=[
        pltpu.VMEM((_C,), jnp.int32),
        pltpu.VMEM((_C, _D), jnp.float32),
        pltpu.SemaphoreType.DMA,
    ],
)
def _gather_kernel(idx_hbm, table_hbm, out_hbm, idx_v, rows_v, sem):
    wid = lax.axis_index("s") * 2 + lax.axis_index("c")
    base = wid * _BPW

    def body(g, carry):
        off = pl.multiple_of(base + g * _C, _C)
        pltpu.sync_copy(idx_hbm.at[pl.ds(off, _C)], idx_v)
        pltpu.async_copy(table_hbm.at[idx_v], rows_v, sem).wait()
        pltpu.sync_copy(rows_v, out_hbm.at[pl.ds(off, _C)])
        return carry

    lax.fori_loop(0, _NCHUNK, body, 0)


def kernel(x, weight):
    idx = x.reshape(-1)
    out = _gather_kernel(idx, weight)
    return out.reshape(x.shape + (weight.shape[1],))


# speedup vs baseline: 1.1137x; 1.0173x over previous
import functools

import jax
import jax.numpy as jnp
from jax import lax
from jax.experimental import pallas as pl
from jax.experimental.pallas import tpu as pltpu
from jax.experimental.pallas import tpu_sc as plsc

_B = 16384 * 50
_D = 32
_NW = 32
_BPW = _B // _NW
_C = 640
_N = _BPW // _C
_NBUF = 4
_DIST = 2

_mesh = plsc.VectorSubcoreMesh(core_axis_name="c", subcore_axis_name="s")


@functools.partial(
    pl.kernel,
    mesh=_mesh,
    out_type=jax.ShapeDtypeStruct((_B, _D), jnp.float32),
    compiler_params=pltpu.CompilerParams(use_tc_tiling_on_sc=False),
    scratch_types=[
        pltpu.VMEM((_BPW,), jnp.int32),
        pltpu.VMEM((_NBUF, _C, _D), jnp.float32),
        pltpu.SemaphoreType.DMA,
        pltpu.SemaphoreType.DMA((_NBUF,)),
        pltpu.SemaphoreType.DMA((_NBUF,)),
    ],
)
def _gather_kernel(idx_hbm, table_hbm, out_hbm, idx_v, rows_v, sem_i,
                   sem_g, sem_o):
    wid = lax.axis_index("s") * 2 + lax.axis_index("c")
    base = pl.multiple_of(wid * _BPW, _BPW)

    pltpu.async_copy(idx_hbm.at[pl.ds(base, _BPW)], idx_v, sem_i).wait()

    def gather_start(g, b):
        pltpu.async_copy(
            table_hbm.at[idx_v.at[pl.ds(g * _C, _C)]],
            rows_v.at[b], sem_g.at[b])

    def gather_wait(g, b):
        pltpu.make_async_copy(
            table_hbm.at[idx_v.at[pl.ds(g * _C, _C)]],
            rows_v.at[b], sem_g.at[b]).wait()

    def out_slice(g):
        return out_hbm.at[pl.ds(pl.multiple_of(base + g * _C, _C), _C)]

    def store_start(g, b):
        pltpu.async_copy(rows_v.at[b], out_slice(g), sem_o.at[b])

    def store_wait(g, b):
        pltpu.make_async_copy(rows_v.at[b], out_slice(g), sem_o.at[b]).wait()

    for g in range(_DIST):
        gather_start(g, g % _NBUF)

    def body(t, carry):
        for b in range(_NBUF):
            g = t * _NBUF + b
            bp = (b + _DIST) % _NBUF
            gather_wait(g, b)
            store_start(g, b)

            @pl.when(g >= _NBUF - _DIST)
            def _():
                store_wait(g - (_NBUF - _DIST), bp)

            @pl.when(g + _DIST < _N)
            def _():
                gather_start(g + _DIST, bp)
        return carry

    lax.fori_loop(0, _N // _NBUF, body, 0)

    for g in range(_N - (_NBUF - _DIST), _N):
        store_wait(g, g % _NBUF)


def kernel(x, weight):
    idx = x.reshape(-1)
    out = _gather_kernel(idx, weight)
    return out.reshape(x.shape + (weight.shape[1],))


# speedup vs baseline: 1.6348x; 1.4679x over previous
import functools

import jax
import jax.numpy as jnp
from jax import lax
from jax.experimental import pallas as pl
from jax.experimental.pallas import tpu as pltpu
from jax.experimental.pallas import tpu_sc as plsc

_V = 1000000
_D = 32
_BATCH = 16384
_S = 50
_NW = 32
_BPW = _BATCH // _NW
_G = _S * (_BPW // 128)

_mesh = plsc.VectorSubcoreMesh(core_axis_name="c", subcore_axis_name="s")


@functools.partial(
    pl.kernel,
    mesh=_mesh,
    out_type=jax.ShapeDtypeStruct((_S * (_D // 8) * (_BATCH // 128), 1024),
                                  jnp.float32),
    compiler_params=pltpu.CompilerParams(use_tc_tiling_on_sc=False,
                                         needs_layout_passes=False),
    scratch_types=[
        pltpu.VMEM((_BPW * _S,), jnp.int32),
        pltpu.VMEM((2, 128), jnp.int32),
        pltpu.VMEM((2, 128, _D), jnp.float32),
        pltpu.VMEM((2, 4096), jnp.float32),
        pltpu.SemaphoreType.DMA,
        pltpu.SemaphoreType.DMA((2,)),
        pltpu.SemaphoreType.DMA((2,)),
    ],
)
def _emb_kernel(idx_hbm, table_hbm, out_hbm, xblk, idxl, rows, tiles,
                sem_x, sem_g, sem_o):
    wid = lax.axis_index("s") * 2 + lax.axis_index("c")
    b0 = wid * _BPW

    pltpu.async_copy(
        idx_hbm.at[pl.ds(pl.multiple_of(b0 * _S, _BPW * _S), _BPW * _S)],
        xblk, sem_x).wait()

    def build_idx(g, b):
        lane50 = lax.iota(jnp.int32, 16) * _S
        s = g // 4
        t2l = g % 4
        base = t2l * 128 * _S + s
        for cb in range(8):
            v = plsc.load_gather(xblk, [base + cb * 16 * _S + lane50])
            idxl[b, pl.ds(cb * 16, 16)] = v

    def gather_start(g, b):
        pltpu.async_copy(table_hbm.at[idxl.at[b]], rows.at[b], sem_g.at[b])

    def gather_wait(g, b):
        pltpu.make_async_copy(table_hbm.at[idxl.at[b]], rows.at[b],
                              sem_g.at[b]).wait()

    def transpose(b):
        def body(j, carry):
            lane = lax.iota(jnp.int32, 16)
            jv = jnp.full((16,), j, jnp.int32)
            for cb in range(8):
                v = plsc.load_gather(rows.at[b], [lane + cb * 16, jv])
                tiles[b, pl.ds(j * 128 + cb * 16, 16)] = v
            return carry
        lax.fori_loop(0, _D, body, 0, unroll=2)

    def row0(g):
        s = g // 4
        t2l = g % 4
        return s * 512 + wid * 4 + t2l

    def store_start(g, b):
        r0 = row0(g)
        for t1 in range(4):
            pltpu.async_copy(tiles.at[b, pl.ds(t1 * 1024, 1024)],
                             out_hbm.at[r0 + t1 * 128], sem_o.at[b])

    def store_wait(g, b):
        r0 = row0(g)
        for t1 in range(4):
            pltpu.make_async_copy(tiles.at[b, pl.ds(t1 * 1024, 1024)],
                                  out_hbm.at[r0 + t1 * 128],
                                  sem_o.at[b]).wait()

    build_idx(0, 0)
    gather_start(0, 0)

    def body(t, carry):
        for b in range(2):
            g = t * 2 + b
            bn = 1 - b
            gather_wait(g, b)

            @pl.when(g + 1 < _G)
            def _():
                build_idx(g + 1, bn)
                gather_start(g + 1, bn)

            @pl.when(g >= 2)
            def _():
                store_wait(g - 2, b)

            transpose(b)
            store_start(g, b)
        return carry

    lax.fori_loop(0, _G // 2, body, 0)

    store_wait(_G - 2, 0)
    store_wait(_G - 1, 1)


def kernel(x, weight):
    idx = x.reshape(-1)
    out2 = _emb_kernel(idx, weight)
    a5 = out2.reshape(_S, _D // 8, _BATCH // 128, 8, 128)
    out = a5.transpose((2, 4, 0, 1, 3)).reshape(_BATCH, _S, _D)
    return out
